# baseline (speedup 1.0000x reference)
import functools

import jax
import jax.numpy as jnp
from jax import lax
from jax.experimental import pallas as pl
from jax.experimental.pallas import tpu as pltpu
from jax.experimental.pallas import tpu_sc as plsc

_BATCH = 16384
_DIM = 2048
_LANES = 16
_NC = 2
_NS = 16
_NW = _NC * _NS
_ROWS_PER_W = _BATCH // _NW
_CHUNK = 8
_NCHUNKS = _ROWS_PER_W // _CHUNK
_GROUPS = _DIM // _LANES
_CHUNK_ELEMS = _CHUNK * _DIM


def _shuffle_body(x_hbm, perm_hbm, out_hbm,
                  perm_v, in_a, in_b, out_a, out_b,
                  sem_ia, sem_ib, sem_oa, sem_ob):
  wid = lax.axis_index("s") * _NC + lax.axis_index("c")
  base = wid * _ROWS_PER_W * _DIM

  pltpu.sync_copy(perm_hbm, perm_v)

  def in_slice(chunk):
    return x_hbm.at[pl.ds(base + chunk * _CHUNK_ELEMS, _CHUNK_ELEMS)]

  def out_slice(chunk):
    return out_hbm.at[pl.ds(base + chunk * _CHUNK_ELEMS, _CHUNK_ELEMS)]

  def compute(in_ref, out_ref):
    @pl.loop(0, _GROUPS)
    def _(g):
      col = g * _LANES
      pv = perm_v[pl.ds(col, _LANES)]
      for r in range(_CHUNK):
        idx = pv + jnp.full((_LANES,), r * _DIM, jnp.int32)
        vals = plsc.load_gather(in_ref, [idx])
        out_ref[pl.ds(r * _DIM + col, _LANES)] = vals

  pltpu.async_copy(in_slice(0), in_a, sem_ia)

  slots = ((in_a, out_a, sem_ia, sem_oa), (in_b, out_b, sem_ib, sem_ob))

  @pl.loop(0, _NCHUNKS, step=2)
  def _(c):
    for s in range(2):
      in_s, out_s, sem_i, sem_o = slots[s]
      in_n, _, sem_in, _ = slots[1 - s]
      chunk = c + s

      @pl.when(chunk + 1 < _NCHUNKS)
      def _():
        pltpu.async_copy(in_slice(chunk + 1), in_n, sem_in)

      pltpu.make_async_copy(in_slice(chunk), in_s, sem_i).wait()

      @pl.when(chunk >= 2)
      def _():
        pltpu.make_async_copy(out_s, out_slice(chunk - 2), sem_o).wait()

      compute(in_s, out_s)
      pltpu.async_copy(out_s, out_slice(chunk), sem_o)

  pltpu.make_async_copy(out_a, out_slice(_NCHUNKS - 2), sem_oa).wait()
  pltpu.make_async_copy(out_b, out_slice(_NCHUNKS - 1), sem_ob).wait()


@jax.jit
def _shuffle(x, perm):
  mesh = plsc.VectorSubcoreMesh(core_axis_name="c", subcore_axis_name="s")
  flat = pl.kernel(
      _shuffle_body,
      out_type=jax.ShapeDtypeStruct((_BATCH * _DIM,), jnp.float32),
      mesh=mesh,
      compiler_params=pltpu.CompilerParams(needs_layout_passes=False),
      scratch_types=[
          pltpu.VMEM((_DIM,), jnp.int32),
          pltpu.VMEM((_CHUNK_ELEMS,), jnp.float32),
          pltpu.VMEM((_CHUNK_ELEMS,), jnp.float32),
          pltpu.VMEM((_CHUNK_ELEMS,), jnp.float32),
          pltpu.VMEM((_CHUNK_ELEMS,), jnp.float32),
          pltpu.SemaphoreType.DMA,
          pltpu.SemaphoreType.DMA,
          pltpu.SemaphoreType.DMA,
          pltpu.SemaphoreType.DMA,
      ],
  )(x.reshape(_BATCH * _DIM), perm)
  return flat.reshape(_BATCH, _DIM)


def kernel(x, perm):
  out = _shuffle(x, perm.astype(jnp.int32))
  log_det = jnp.zeros((x.shape[0],), dtype=x.dtype)
  return (out, log_det)


# speedup vs baseline: 1.6141x; 1.6141x over previous
import functools

import jax
import jax.numpy as jnp
from jax import lax
from jax.experimental import pallas as pl
from jax.experimental.pallas import tpu as pltpu
from jax.experimental.pallas import tpu_sc as plsc

_BATCH = 16384
_DIM = 2048
_LANES = 16
_NC = 2
_NS = 16
_NW = _NC * _NS
_ROWS_PER_W = _BATCH // _NW
_CHUNK = 8
_NCHUNKS = _ROWS_PER_W // _CHUNK
_GROUPS = _DIM // _LANES
_CHUNK_ELEMS = _CHUNK * _DIM


def _shuffle_body(x_hbm, perm_hbm, out_hbm,
                  perm_v, in_a, in_b, out_a, out_b,
                  sem_ia, sem_ib, sem_oa, sem_ob):
  wid = lax.axis_index("s") * _NC + lax.axis_index("c")
  base = wid * _ROWS_PER_W

  pltpu.sync_copy(perm_hbm, perm_v)

  def in_slice(chunk):
    return x_hbm.at[pl.ds(base + chunk * _CHUNK, _CHUNK)]

  def out_slice(chunk):
    return out_hbm.at[pl.ds(base + chunk * _CHUNK, _CHUNK)]

  def compute(in_ref, out_ref):
    @pl.loop(0, _GROUPS)
    def _(g):
      col = g * _LANES
      pv = perm_v[pl.ds(col, _LANES)]
      for r in range(_CHUNK):
        ridx = jnp.full((_LANES,), r, jnp.int32)
        vals = plsc.load_gather(in_ref, [ridx, pv])
        out_ref[r, pl.ds(col, _LANES)] = vals

  pltpu.async_copy(in_slice(0), in_a, sem_ia)

  slots = ((in_a, out_a, sem_ia, sem_oa), (in_b, out_b, sem_ib, sem_ob))

  @pl.loop(0, _NCHUNKS, step=2)
  def _(c):
    for s in range(2):
      in_s, out_s, sem_i, sem_o = slots[s]
      in_n, _, sem_in, _ = slots[1 - s]
      chunk = c + s

      @pl.when(chunk + 1 < _NCHUNKS)
      def _():
        pltpu.async_copy(in_slice(chunk + 1), in_n, sem_in)

      pltpu.make_async_copy(in_slice(chunk), in_s, sem_i).wait()

      @pl.when(chunk >= 2)
      def _():
        pltpu.make_async_copy(out_s, out_slice(chunk - 2), sem_o).wait()

      compute(in_s, out_s)
      pltpu.async_copy(out_s, out_slice(chunk), sem_o)

  pltpu.make_async_copy(out_a, out_slice(_NCHUNKS - 2), sem_oa).wait()
  pltpu.make_async_copy(out_b, out_slice(_NCHUNKS - 1), sem_ob).wait()


@jax.jit
def _shuffle(x, perm):
  mesh = plsc.VectorSubcoreMesh(core_axis_name="c", subcore_axis_name="s")
  flat = pl.kernel(
      _shuffle_body,
      out_type=jax.ShapeDtypeStruct((_BATCH, _DIM), jnp.float32),
      mesh=mesh,
      compiler_params=pltpu.CompilerParams(needs_layout_passes=False),
      scratch_types=[
          pltpu.VMEM((_DIM,), jnp.int32),
          pltpu.VMEM((_CHUNK, _DIM), jnp.float32),
          pltpu.VMEM((_CHUNK, _DIM), jnp.float32),
          pltpu.VMEM((_CHUNK, _DIM), jnp.float32),
          pltpu.VMEM((_CHUNK, _DIM), jnp.float32),
          pltpu.SemaphoreType.DMA,
          pltpu.SemaphoreType.DMA,
          pltpu.SemaphoreType.DMA,
          pltpu.SemaphoreType.DMA,
      ],
  )(x, perm)
  return flat


def kernel(x, perm):
  out = _shuffle(x, perm.astype(jnp.int32))
  log_det = jnp.zeros((x.shape[0],), dtype=x.dtype)
  return (out, log_det)
